# baseline (speedup 1.0000x reference)
import functools

import jax
import jax.numpy as jnp
from jax import lax
from jax.experimental import pallas as pl
from jax.experimental.pallas import tpu as pltpu
from jax.experimental.pallas import tpu_sc as plsc

CH = 112
XCH = 64

_SC_PARAMS = pltpu.CompilerParams(needs_layout_passes=False,
                                  use_tc_tiling_on_sc=False)


def _sc_mesh():
    return plsc.VectorSubcoreMesh(core_axis_name="c", subcore_axis_name="s")


def _emb_deg_kernel(V, D, N_pad, K0, K1, NC, NS):
    NW = NC * NS
    rows_w = N_pad // NW
    nx = rows_w // XCH
    rows_s = N_pad // NS

    @functools.partial(
        pl.kernel,
        out_type=[
            jax.ShapeDtypeStruct((N_pad, D), jnp.float32),
            jax.ShapeDtypeStruct((NC, N_pad, 16), jnp.float32),
        ],
        mesh=_sc_mesh(),
        compiler_params=_SC_PARAMS,
        scratch_types=[
            pltpu.VMEM((nx, XCH), jnp.int32),
            pltpu.VMEM((4, XCH, D), jnp.float32),
            pltpu.VMEM((2, 6, CH), jnp.int32),
            pltpu.VMEM((CH, 16), jnp.float32),
            pltpu.VMEM((CH, 16), jnp.float32),
            pltpu.VMEM_SHARED((N_pad, 16), jnp.float32),
            pltpu.SemaphoreType.DMA((4,)),
            pltpu.SemaphoreType.DMA((4,)),
            pltpu.SemaphoreType.DMA((2,)),
            pltpu.SemaphoreType.DMA,
            pltpu.SemaphoreType.DMA,
        ],
    )
    def k(emb_h, xi_h, pk_h, z16_h, h0_h, deg_h,
          xi_v, ebuf, pk_v, w0, w1, deg_sh, egs, ews, pks, ds0, ds1):
        c = lax.axis_index("c")
        s = lax.axis_index("s")
        wid = s * NC + c
        start = jnp.where(c == 0, s * K0, NS * K0 + s * K1)
        cnt = jnp.where(c == 0, K0, K1)
        sl = pl.ds(s * rows_s, rows_s)
        pltpu.sync_copy(z16_h.at[sl], deg_sh.at[sl])
        pltpu.sync_copy(xi_h.at[wid], xi_v)

        def eg(j):
            return pltpu.make_async_copy(emb_h.at[xi_v.at[j]],
                                         ebuf.at[j % 4], egs.at[j % 4])

        def ew(j):
            return pltpu.make_async_copy(
                ebuf.at[j % 4], h0_h.at[pl.ds(wid * rows_w + j * XCH, XCH)],
                ews.at[j % 4])

        def pkd(p, b):
            return pltpu.make_async_copy(pk_h.at[start + p], pk_v.at[b],
                                         pks.at[b])

        def dsc(b, par, wbuf, sem):
            return pltpu.make_async_copy(wbuf, deg_sh.at[pk_v.at[b, 2 + par]],
                                         sem)

        def build(b, par, wbuf):
            bs = lax.broadcast_in_dim(b, (16,), ())
            ws = lax.broadcast_in_dim(4 + par, (16,), ())

            def body(i, carry):
                wi = plsc.load_gather(
                    pk_v, [bs, ws, lax.broadcast_in_dim(i, (16,), ())])
                wbuf[i, :] = plsc.bitcast(wi, jnp.float32)
                return carry

            lax.fori_loop(0, CH, body, 0, unroll=4)

        pkd(0, 0).start()
        for j in range(min(nx, 4)):
            eg(j).start()
        for j in range(nx):
            eg(j).wait()
            ew(j).start()
            if j + 4 < nx:
                ew(j).wait()
                eg(j + 4).start()
        for j in range(max(nx - 4, 0), nx):
            ew(j).wait()

        pkd(0, 0).wait()
        plsc.subcore_barrier()

        def pair(p, carry):
            b = p & 1
            nb = 1 - b

            @pl.when(p + 1 < cnt)
            def _():
                pkd(p + 1, nb).start()

            build(b, 0, w0)
            dsc(b, 0, w0, ds0).start(add=True)
            build(b, 1, w1)
            dsc(b, 1, w1, ds1).start(add=True)

            @pl.when(p + 1 < cnt)
            def _():
                pkd(p + 1, nb).wait()

            dsc(b, 0, w0, ds0).wait()
            dsc(b, 1, w1, ds1).wait()
            return carry

        lax.fori_loop(0, cnt, pair, 0)
        plsc.subcore_barrier()
        pltpu.sync_copy(deg_sh.at[sl], deg_h.at[c].at[sl])

    return k


def _agg_kernel(Dc, N_pad, K0, K1, NC, NS):
    rows_s = N_pad // NS

    @functools.partial(
        pl.kernel,
        out_type=jax.ShapeDtypeStruct((NC, N_pad, Dc), jnp.float32),
        mesh=_sc_mesh(),
        compiler_params=_SC_PARAMS,
        scratch_types=[
            pltpu.VMEM((2, 6, CH), jnp.int32),
            pltpu.VMEM((CH, Dc), jnp.float32),
            pltpu.VMEM((CH, Dc), jnp.float32),
            pltpu.VMEM_SHARED((N_pad, Dc), jnp.float32),
            pltpu.SemaphoreType.DMA((2,)),
            pltpu.SemaphoreType.DMA,
            pltpu.SemaphoreType.DMA,
            pltpu.SemaphoreType.DMA,
            pltpu.SemaphoreType.DMA,
        ],
    )
    def k(g_h, pk_h, acc_h, pk_v, buf0, buf1, acc_sh, pks, gs0, gs1, ss0, ss1):
        c = lax.axis_index("c")
        s = lax.axis_index("s")
        start = jnp.where(c == 0, s * K0, NS * K0 + s * K1)
        cnt = jnp.where(c == 0, K0, K1)
        sl = pl.ds(s * rows_s, rows_s)
        pltpu.sync_copy(g_h.at[sl], acc_sh.at[sl])
        plsc.subcore_barrier()

        def pkd(p, b):
            return pltpu.make_async_copy(pk_h.at[start + p], pk_v.at[b],
                                         pks.at[b])

        def gat(b, par, buf, sem):
            return pltpu.make_async_copy(g_h.at[pk_v.at[b, par]], buf, sem)

        def sca(b, par, buf, sem):
            return pltpu.make_async_copy(buf, acc_sh.at[pk_v.at[b, 2 + par]],
                                         sem)

        def scale(b, par, buf):
            bs = lax.broadcast_in_dim(b, (16,), ())
            ws = lax.broadcast_in_dim(4 + par, (16,), ())

            def body(i, carry):
                wi = plsc.load_gather(
                    pk_v, [bs, ws, lax.broadcast_in_dim(i, (16,), ())])
                wb = plsc.bitcast(wi, jnp.float32)
                for kk in range(Dc // 16):
                    csl = pl.ds(kk * 16, 16)
                    buf[i, csl] = buf[i, csl] * wb
                return carry

            lax.fori_loop(0, CH, body, 0, unroll=2)

        pkd(0, 0).start()
        pkd(0, 0).wait()
        gat(0, 0, buf0, gs0).start()
        gat(0, 1, buf1, gs1).start()

        def pair(p, carry):
            b = p & 1
            nb = 1 - b

            @pl.when(p + 1 < cnt)
            def _():
                pkd(p + 1, nb).start()

            gat(b, 0, buf0, gs0).wait()
            scale(b, 0, buf0)
            sca(b, 0, buf0, ss0).start(add=True)
            gat(b, 1, buf1, gs1).wait()
            scale(b, 1, buf1)
            sca(b, 1, buf1, ss1).start(add=True)

            @pl.when(p + 1 < cnt)
            def _():
                pkd(p + 1, nb).wait()
                sca(b, 0, buf0, ss0).wait()
                gat(nb, 0, buf0, gs0).start()
                sca(b, 1, buf1, ss1).wait()
                gat(nb, 1, buf1, gs1).start()

            @pl.when(p + 1 >= cnt)
            def _():
                sca(b, 0, buf0, ss0).wait()
                sca(b, 1, buf1, ss1).wait()

            return carry

        lax.fori_loop(0, cnt, pair, 0)
        plsc.subcore_barrier()
        pltpu.sync_copy(acc_sh.at[sl], acc_h.at[c].at[sl])

    return k


def _agg_kernel_bf16(N_pad, K0, K1, NC, NS):
    Dc = 128
    rows_s = N_pad // NS

    @functools.partial(
        pl.kernel,
        out_type=jax.ShapeDtypeStruct((NC, N_pad, Dc), jnp.float32),
        mesh=_sc_mesh(),
        compiler_params=_SC_PARAMS,
        scratch_types=[
            pltpu.VMEM((2, 6, CH), jnp.int32),
            pltpu.VMEM((CH, 64), jnp.int32),
            pltpu.VMEM((CH, 64), jnp.int32),
            pltpu.VMEM((CH, Dc), jnp.float32),
            pltpu.VMEM((CH, Dc), jnp.float32),
            pltpu.VMEM_SHARED((N_pad, Dc), jnp.float32),
            pltpu.SemaphoreType.DMA((2,)),
            pltpu.SemaphoreType.DMA,
            pltpu.SemaphoreType.DMA,
            pltpu.SemaphoreType.DMA,
            pltpu.SemaphoreType.DMA,
        ],
    )
    def k(gi_h, gsig_h, pk_h, acc_h, pk_v, gb0, gb1, sb0, sb1, acc_sh,
          pks, gs0, gs1, ss0, ss1):
        c = lax.axis_index("c")
        s = lax.axis_index("s")
        start = jnp.where(c == 0, s * K0, NS * K0 + s * K1)
        cnt = jnp.where(c == 0, K0, K1)
        sl = pl.ds(s * rows_s, rows_s)
        pltpu.sync_copy(gsig_h.at[sl], acc_sh.at[sl])
        plsc.subcore_barrier()

        def pkd(p, b):
            return pltpu.make_async_copy(pk_h.at[start + p], pk_v.at[b],
                                         pks.at[b])

        def gat(b, par, gb, sem):
            return pltpu.make_async_copy(gi_h.at[pk_v.at[b, par]], gb, sem)

        def sca(b, par, sb, sem):
            return pltpu.make_async_copy(sb, acc_sh.at[pk_v.at[b, 2 + par]],
                                         sem)

        def scale(b, par, gb, sb):
            bs = lax.broadcast_in_dim(b, (16,), ())
            ws = lax.broadcast_in_dim(4 + par, (16,), ())
            himask = jnp.full((16,), -65536, jnp.int32)

            def body(i, carry):
                wi = plsc.load_gather(
                    pk_v, [bs, ws, lax.broadcast_in_dim(i, (16,), ())])
                wb = plsc.bitcast(wi, jnp.float32)
                for m in range(4):
                    v = gb[i, pl.ds(m * 16, 16)]
                    ev = plsc.bitcast(lax.shift_left(v, 16), jnp.float32)
                    od = plsc.bitcast(lax.bitwise_and(v, himask), jnp.float32)
                    sb[i, pl.ds(m * 32, 16)] = ev * wb
                    sb[i, pl.ds(m * 32 + 16, 16)] = od * wb
                return carry

            lax.fori_loop(0, CH, body, 0, unroll=2)

        pkd(0, 0).start()
        pkd(0, 0).wait()
        gat(0, 0, gb0, gs0).start()
        gat(0, 1, gb1, gs1).start()

        def pair(p, carry):
            b = p & 1
            nb = 1 - b

            gat(b, 0, gb0, gs0).wait()

            @pl.when(p >= 1)
            def _():
                sca(nb, 0, sb0, ss0).wait()
                sca(nb, 1, sb1, ss1).wait()

            @pl.when(p + 1 < cnt)
            def _():
                pkd(p + 1, nb).start()

            scale(b, 0, gb0, sb0)
            sca(b, 0, sb0, ss0).start(add=True)

            @pl.when(p + 1 < cnt)
            def _():
                pkd(p + 1, nb).wait()
                gat(nb, 0, gb0, gs0).start()

            gat(b, 1, gb1, gs1).wait()
            scale(b, 1, gb1, sb1)
            sca(b, 1, sb1, ss1).start(add=True)

            @pl.when(p + 1 < cnt)
            def _():
                gat(nb, 1, gb1, gs1).start()

            return carry

        lax.fori_loop(0, cnt, pair, 0)
        lb = (cnt - 1) & 1
        sca(lb, 0, sb0, ss0).wait()
        sca(lb, 1, sb1, ss1).wait()
        plsc.subcore_barrier()
        pltpu.sync_copy(acc_sh.at[sl], acc_h.at[c].at[sl])

    return k


def _dinv(deg_ref):
    deg = 1.0 + deg_ref[0][:, 0:1] + deg_ref[1][:, 0:1]
    return jnp.where(deg > 0, lax.rsqrt(deg), 0.0)


def _lin1_body(deg_ref, h0_ref, w1_ref, w1s_ref, g1b_ref, g1s_ref):
    dinv = _dinv(deg_ref)
    h0 = h0_ref[...]
    hw = jnp.dot(h0, w1_ref[...], preferred_element_type=jnp.float32)
    g1b_ref[...] = (dinv * hw).astype(jnp.bfloat16)
    hws = jnp.dot(h0, w1s_ref[...], preferred_element_type=jnp.float32)
    g1s_ref[...] = dinv * hws


def _lin2_body(acc_ref, g1_ref, deg_ref, b1_ref, w2_ref, g2_ref):
    dinv = _dinv(deg_ref)
    a = acc_ref[0] + acc_ref[1] - g1_ref[...]
    pre = dinv * a + b1_ref[...]
    h1 = jnp.where(pre >= 0, pre, 0.01 * pre)
    hw = jnp.dot(h1, w2_ref[...], preferred_element_type=jnp.float32)
    g2_ref[...] = dinv * hw


def _final_body(acc_ref, g2_ref, deg_ref, b2_ref, out_ref):
    dinv = _dinv(deg_ref)
    z = dinv * (acc_ref[0] + acc_ref[1] - g2_ref[...]) + b2_ref[...]
    valid = lax.broadcasted_iota(jnp.int32, z.shape, 1) < 40
    zm = jnp.where(valid, z, -1e30)
    m = jnp.max(zm, axis=1, keepdims=True)
    e = jnp.where(valid, jnp.exp(zm - m), 0.0)
    ssum = jnp.sum(e, axis=1, keepdims=True)
    out_ref[...] = z - m - jnp.log(ssum)


def kernel(x, edge_index, edge_attr, emb, W1, b1, W2, b2):
    N = x.shape[0]
    E = edge_index.shape[1]
    V, D = emb.shape
    C = W2.shape[1]
    Dc2 = 48

    info = plsc.get_sparse_core_info()
    NC, NS = info.num_cores, info.num_subcores
    NW = NC * NS

    N_pad = -(-N // (NW * XCH)) * (NW * XCH)
    PAIR = 2 * CH
    npl_min = -(-E // PAIR)
    KSUM = -(-npl_min // NS)
    NPL = NS * KSUM
    E_pad = NPL * PAIR

    def _split(fr0):
        k0 = min(KSUM - 1, max(1, round(fr0 * KSUM)))
        return k0, KSUM - k0

    K0e, K1e = _split(0.60)
    K0a, K1a = _split(0.75)
    K0b, K1b = _split(0.67)

    xi = jnp.concatenate([x[:, 0].astype(jnp.int32),
                          jnp.zeros((N_pad - N,), jnp.int32)])
    xi = xi.reshape(NW, (N_pad // NW) // XCH, XCH)
    epad = E_pad - E
    rowp = jnp.concatenate([edge_index[0].astype(jnp.int32),
                            jnp.zeros((epad,), jnp.int32)])
    colp = jnp.concatenate([edge_index[1].astype(jnp.int32),
                            jnp.zeros((epad,), jnp.int32)])
    wp = jnp.concatenate([edge_attr, jnp.zeros((epad,), jnp.float32)])
    wbits = lax.bitcast_convert_type(wp, jnp.int32)
    pk = jnp.concatenate([rowp.reshape(NPL, 2, CH),
                          colp.reshape(NPL, 2, CH),
                          wbits.reshape(NPL, 2, CH)], axis=1)
    z16 = jnp.zeros((N_pad, 16), jnp.float32)

    h0, deg2 = _emb_deg_kernel(V, D, N_pad, K0e, K1e, NC, NS)(emb, xi, pk, z16)

    sigma = jnp.asarray(
        [32 * m + o for m in range(D // 32)
         for o in list(range(0, 32, 2)) + list(range(1, 32, 2))], jnp.int32)
    W1s = W1[:, sigma]

    nblk = N_pad // 256
    deg_spec = pl.BlockSpec((NC, 256, 16), lambda i: (0, i, 0))
    g1b, g1s = pl.pallas_call(
        _lin1_body,
        grid=(nblk,),
        in_specs=[deg_spec,
                  pl.BlockSpec((256, D), lambda i: (i, 0)),
                  pl.BlockSpec((D, D), lambda i: (0, 0)),
                  pl.BlockSpec((D, D), lambda i: (0, 0))],
        out_specs=[pl.BlockSpec((256, D), lambda i: (i, 0)),
                   pl.BlockSpec((256, D), lambda i: (i, 0))],
        out_shape=[jax.ShapeDtypeStruct((N_pad, D), jnp.bfloat16),
                   jax.ShapeDtypeStruct((N_pad, D), jnp.float32)],
    )(deg2, h0, W1, W1s)
    g1i = lax.bitcast_convert_type(g1b.reshape(N_pad, D // 2, 2), jnp.int32)

    acc1 = _agg_kernel_bf16(N_pad, K0a, K1a, NC, NS)(g1i, g1s, pk)

    W2p = jnp.pad(W2, ((0, 0), (0, Dc2 - C)))[sigma, :]
    b1 = b1[sigma]
    g2 = pl.pallas_call(
        _lin2_body,
        grid=(nblk,),
        in_specs=[pl.BlockSpec((NC, 256, D), lambda i: (0, i, 0)),
                  pl.BlockSpec((256, D), lambda i: (i, 0)),
                  deg_spec,
                  pl.BlockSpec((1, D), lambda i: (0, 0)),
                  pl.BlockSpec((D, Dc2), lambda i: (0, 0))],
        out_specs=pl.BlockSpec((256, Dc2), lambda i: (i, 0)),
        out_shape=jax.ShapeDtypeStruct((N_pad, Dc2), jnp.float32),
    )(acc1, g1s, deg2, b1[None, :], W2p)

    acc2 = _agg_kernel(Dc2, N_pad, K0b, K1b, NC, NS)(g2, pk)

    b2p = jnp.pad(b2, (0, Dc2 - C))
    out = pl.pallas_call(
        _final_body,
        grid=(nblk,),
        in_specs=[pl.BlockSpec((NC, 256, Dc2), lambda i: (0, i, 0)),
                  pl.BlockSpec((256, Dc2), lambda i: (i, 0)),
                  deg_spec,
                  pl.BlockSpec((1, Dc2), lambda i: (0, 0))],
        out_specs=pl.BlockSpec((256, Dc2), lambda i: (i, 0)),
        out_shape=jax.ShapeDtypeStruct((N_pad, Dc2), jnp.float32),
    )(acc2, g2, deg2, b2p[None, :])

    return out[:N, :C]


# speedup vs baseline: 1.4140x; 1.4140x over previous
import functools

import jax
import jax.numpy as jnp
from jax import lax
from jax.experimental import pallas as pl
from jax.experimental.pallas import tpu as pltpu
from jax.experimental.pallas import tpu_sc as plsc

CH = 128
XCH = 64

_SC_PARAMS = pltpu.CompilerParams(needs_layout_passes=False,
                                  use_tc_tiling_on_sc=False)


def _sc_mesh():
    return plsc.VectorSubcoreMesh(core_axis_name="c", subcore_axis_name="s")


def _emb_deg_kernel(V, D, N_pad, K0, K1, NC, NS):
    NW = NC * NS
    rows_w = N_pad // NW
    nx = rows_w // XCH
    rows_s = N_pad // NS

    @functools.partial(
        pl.kernel,
        out_type=[
            jax.ShapeDtypeStruct((N_pad, D), jnp.float32),
            jax.ShapeDtypeStruct((NC, N_pad, 16), jnp.float32),
        ],
        mesh=_sc_mesh(),
        compiler_params=_SC_PARAMS,
        scratch_types=[
            pltpu.VMEM((nx, XCH), jnp.int32),
            pltpu.VMEM((4, XCH, D), jnp.float32),
            pltpu.VMEM((2, 6, CH), jnp.int32),
            pltpu.VMEM((CH, 16), jnp.float32),
            pltpu.VMEM((CH, 16), jnp.float32),
            pltpu.VMEM_SHARED((N_pad, 16), jnp.float32),
            pltpu.SemaphoreType.DMA((4,)),
            pltpu.SemaphoreType.DMA((4,)),
            pltpu.SemaphoreType.DMA((2,)),
            pltpu.SemaphoreType.DMA,
            pltpu.SemaphoreType.DMA,
        ],
    )
    def k(emb_h, xi_h, pk_h, z16_h, h0_h, deg_h,
          xi_v, ebuf, pk_v, w0, w1, deg_sh, egs, ews, pks, ds0, ds1):
        c = lax.axis_index("c")
        s = lax.axis_index("s")
        wid = s * NC + c
        start = jnp.where(c == 0, s * K0, NS * K0 + s * K1)
        cnt = jnp.where(c == 0, K0, K1)
        sl = pl.ds(s * rows_s, rows_s)
        pltpu.sync_copy(z16_h.at[sl], deg_sh.at[sl])
        pltpu.sync_copy(xi_h.at[wid], xi_v)

        def eg(j):
            return pltpu.make_async_copy(emb_h.at[xi_v.at[j]],
                                         ebuf.at[j % 4], egs.at[j % 4])

        def ew(j):
            return pltpu.make_async_copy(
                ebuf.at[j % 4], h0_h.at[pl.ds(wid * rows_w + j * XCH, XCH)],
                ews.at[j % 4])

        def pkd(p, b):
            return pltpu.make_async_copy(pk_h.at[start + p], pk_v.at[b],
                                         pks.at[b])

        def dsc(b, par, wbuf, sem):
            return pltpu.make_async_copy(wbuf, deg_sh.at[pk_v.at[b, 2 + par]],
                                         sem)

        def build(b, par, wbuf):
            bs = lax.broadcast_in_dim(b, (16,), ())
            ws = lax.broadcast_in_dim(4 + par, (16,), ())

            def body(i, carry):
                wi = plsc.load_gather(
                    pk_v, [bs, ws, lax.broadcast_in_dim(i, (16,), ())])
                wbuf[i, :] = plsc.bitcast(wi, jnp.float32)
                return carry

            lax.fori_loop(0, CH, body, 0, unroll=4)

        pkd(0, 0).start()
        for j in range(min(nx, 4)):
            eg(j).start()
        for j in range(nx):
            eg(j).wait()
            ew(j).start()
            if j + 4 < nx:
                ew(j).wait()
                eg(j + 4).start()
        for j in range(max(nx - 4, 0), nx):
            ew(j).wait()

        pkd(0, 0).wait()
        plsc.subcore_barrier()

        def pair(p, carry):
            b = p & 1
            nb = 1 - b

            @pl.when(p + 1 < cnt)
            def _():
                pkd(p + 1, nb).start()

            build(b, 0, w0)
            dsc(b, 0, w0, ds0).start(add=True)
            build(b, 1, w1)
            dsc(b, 1, w1, ds1).start(add=True)

            @pl.when(p + 1 < cnt)
            def _():
                pkd(p + 1, nb).wait()

            dsc(b, 0, w0, ds0).wait()
            dsc(b, 1, w1, ds1).wait()
            return carry

        lax.fori_loop(0, cnt, pair, 0)
        plsc.subcore_barrier()
        pltpu.sync_copy(deg_sh.at[sl], deg_h.at[c].at[sl])

    return k


def _agg_kernel(Dc, N_pad, K0, K1, NC, NS):
    rows_s = N_pad // NS

    @functools.partial(
        pl.kernel,
        out_type=jax.ShapeDtypeStruct((NC, N_pad, Dc), jnp.float32),
        mesh=_sc_mesh(),
        compiler_params=_SC_PARAMS,
        scratch_types=[
            pltpu.VMEM((2, 6, CH), jnp.int32),
            pltpu.VMEM((CH, Dc), jnp.float32),
            pltpu.VMEM((CH, Dc), jnp.float32),
            pltpu.VMEM_SHARED((N_pad, Dc), jnp.float32),
            pltpu.SemaphoreType.DMA((2,)),
            pltpu.SemaphoreType.DMA,
            pltpu.SemaphoreType.DMA,
            pltpu.SemaphoreType.DMA,
            pltpu.SemaphoreType.DMA,
        ],
    )
    def k(g_h, pk_h, acc_h, pk_v, buf0, buf1, acc_sh, pks, gs0, gs1, ss0, ss1):
        c = lax.axis_index("c")
        s = lax.axis_index("s")
        start = jnp.where(c == 0, s * K0, NS * K0 + s * K1)
        cnt = jnp.where(c == 0, K0, K1)
        sl = pl.ds(s * rows_s, rows_s)
        pltpu.sync_copy(g_h.at[sl], acc_sh.at[sl])
        plsc.subcore_barrier()

        def pkd(p, b):
            return pltpu.make_async_copy(pk_h.at[start + p], pk_v.at[b],
                                         pks.at[b])

        def gat(b, par, buf, sem):
            return pltpu.make_async_copy(g_h.at[pk_v.at[b, par]], buf, sem)

        def sca(b, par, buf, sem):
            return pltpu.make_async_copy(buf, acc_sh.at[pk_v.at[b, 2 + par]],
                                         sem)

        def scale(b, par, buf):
            bs = lax.broadcast_in_dim(b, (16,), ())
            ws = lax.broadcast_in_dim(4 + par, (16,), ())

            def body(i, carry):
                wi = plsc.load_gather(
                    pk_v, [bs, ws, lax.broadcast_in_dim(i, (16,), ())])
                wb = plsc.bitcast(wi, jnp.float32)
                for kk in range(Dc // 16):
                    csl = pl.ds(kk * 16, 16)
                    buf[i, csl] = buf[i, csl] * wb
                return carry

            lax.fori_loop(0, CH, body, 0, unroll=2)

        pkd(0, 0).start()
        pkd(0, 0).wait()
        gat(0, 0, buf0, gs0).start()
        gat(0, 1, buf1, gs1).start()

        def pair(p, carry):
            b = p & 1
            nb = 1 - b

            @pl.when(p + 1 < cnt)
            def _():
                pkd(p + 1, nb).start()

            gat(b, 0, buf0, gs0).wait()
            scale(b, 0, buf0)
            sca(b, 0, buf0, ss0).start(add=True)
            gat(b, 1, buf1, gs1).wait()
            scale(b, 1, buf1)
            sca(b, 1, buf1, ss1).start(add=True)

            @pl.when(p + 1 < cnt)
            def _():
                pkd(p + 1, nb).wait()
                sca(b, 0, buf0, ss0).wait()
                gat(nb, 0, buf0, gs0).start()
                sca(b, 1, buf1, ss1).wait()
                gat(nb, 1, buf1, gs1).start()

            @pl.when(p + 1 >= cnt)
            def _():
                sca(b, 0, buf0, ss0).wait()
                sca(b, 1, buf1, ss1).wait()

            return carry

        lax.fori_loop(0, cnt, pair, 0)
        plsc.subcore_barrier()
        pltpu.sync_copy(acc_sh.at[sl], acc_h.at[c].at[sl])

    return k


def _dinv(deg_ref):
    deg = 1.0 + deg_ref[0][:, 0:1] + deg_ref[1][:, 0:1]
    return jnp.where(deg > 0, lax.rsqrt(deg), 0.0)


def _lin1_body(deg_ref, h0_ref, w1_ref, g1_ref):
    dinv = _dinv(deg_ref)
    hw = jnp.dot(h0_ref[...], w1_ref[...], preferred_element_type=jnp.float32)
    g1_ref[...] = dinv * hw


def _lin2_body(acc_ref, g1_ref, deg_ref, b1_ref, w2_ref, g2_ref):
    dinv = _dinv(deg_ref)
    a = acc_ref[0] + acc_ref[1] - g1_ref[...]
    pre = dinv * a + b1_ref[...]
    h1 = jnp.where(pre >= 0, pre, 0.01 * pre)
    hw = jnp.dot(h1, w2_ref[...], preferred_element_type=jnp.float32)
    g2_ref[...] = dinv * hw


def _final_body(acc_ref, g2_ref, deg_ref, b2_ref, out_ref):
    dinv = _dinv(deg_ref)
    z = dinv * (acc_ref[0] + acc_ref[1] - g2_ref[...]) + b2_ref[...]
    valid = lax.broadcasted_iota(jnp.int32, z.shape, 1) < 40
    zm = jnp.where(valid, z, -1e30)
    m = jnp.max(zm, axis=1, keepdims=True)
    e = jnp.where(valid, jnp.exp(zm - m), 0.0)
    ssum = jnp.sum(e, axis=1, keepdims=True)
    out_ref[...] = z - m - jnp.log(ssum)


def kernel(x, edge_index, edge_attr, emb, W1, b1, W2, b2):
    N = x.shape[0]
    E = edge_index.shape[1]
    V, D = emb.shape
    C = W2.shape[1]
    Dc2 = 48

    info = plsc.get_sparse_core_info()
    NC, NS = info.num_cores, info.num_subcores
    NW = NC * NS

    N_pad = -(-N // (NW * XCH)) * (NW * XCH)
    PAIR = 2 * CH
    npl_min = -(-E // PAIR)
    KSUM = -(-npl_min // NS)
    NPL = NS * KSUM
    E_pad = NPL * PAIR

    def _split(fr0):
        k0 = min(KSUM - 1, max(1, round(fr0 * KSUM)))
        return k0, KSUM - k0

    K0e, K1e = _split(0.60)
    K0a, K1a = _split(0.75)
    K0b, K1b = _split(0.67)

    xi = jnp.concatenate([x[:, 0].astype(jnp.int32),
                          jnp.zeros((N_pad - N,), jnp.int32)])
    xi = xi.reshape(NW, (N_pad // NW) // XCH, XCH)
    epad = E_pad - E
    rowp = jnp.concatenate([edge_index[0].astype(jnp.int32),
                            jnp.zeros((epad,), jnp.int32)])
    colp = jnp.concatenate([edge_index[1].astype(jnp.int32),
                            jnp.zeros((epad,), jnp.int32)])
    wp = jnp.concatenate([edge_attr, jnp.zeros((epad,), jnp.float32)])
    wbits = lax.bitcast_convert_type(wp, jnp.int32)
    pk = jnp.concatenate([rowp.reshape(NPL, 2, CH),
                          colp.reshape(NPL, 2, CH),
                          wbits.reshape(NPL, 2, CH)], axis=1)
    z16 = jnp.zeros((N_pad, 16), jnp.float32)

    h0, deg2 = _emb_deg_kernel(V, D, N_pad, K0e, K1e, NC, NS)(emb, xi, pk, z16)

    nblk = N_pad // 512
    deg_spec = pl.BlockSpec((NC, 512, 16), lambda i: (0, i, 0))
    g1 = pl.pallas_call(
        _lin1_body,
        grid=(nblk,),
        in_specs=[deg_spec,
                  pl.BlockSpec((512, D), lambda i: (i, 0)),
                  pl.BlockSpec((D, D), lambda i: (0, 0))],
        out_specs=pl.BlockSpec((512, D), lambda i: (i, 0)),
        out_shape=jax.ShapeDtypeStruct((N_pad, D), jnp.float32),
    )(deg2, h0, W1)

    acc1 = _agg_kernel(D, N_pad, K0a, K1a, NC, NS)(g1, pk)

    W2p = jnp.pad(W2, ((0, 0), (0, Dc2 - C)))
    g2 = pl.pallas_call(
        _lin2_body,
        grid=(nblk,),
        in_specs=[pl.BlockSpec((NC, 512, D), lambda i: (0, i, 0)),
                  pl.BlockSpec((512, D), lambda i: (i, 0)),
                  deg_spec,
                  pl.BlockSpec((1, D), lambda i: (0, 0)),
                  pl.BlockSpec((D, Dc2), lambda i: (0, 0))],
        out_specs=pl.BlockSpec((512, Dc2), lambda i: (i, 0)),
        out_shape=jax.ShapeDtypeStruct((N_pad, Dc2), jnp.float32),
    )(acc1, g1, deg2, b1[None, :], W2p)

    acc2 = _agg_kernel(Dc2, N_pad, K0b, K1b, NC, NS)(g2, pk)

    b2p = jnp.pad(b2, (0, Dc2 - C))
    out = pl.pallas_call(
        _final_body,
        grid=(nblk,),
        in_specs=[pl.BlockSpec((NC, 512, Dc2), lambda i: (0, i, 0)),
                  pl.BlockSpec((512, Dc2), lambda i: (i, 0)),
                  deg_spec,
                  pl.BlockSpec((1, Dc2), lambda i: (0, 0))],
        out_specs=pl.BlockSpec((512, Dc2), lambda i: (i, 0)),
        out_shape=jax.ShapeDtypeStruct((N_pad, Dc2), jnp.float32),
    )(acc2, g2, deg2, b2p[None, :])

    return out[:N, :C]


# speedup vs baseline: 1.4382x; 1.0171x over previous
import functools

import jax
import jax.numpy as jnp
from jax import lax
from jax.experimental import pallas as pl
from jax.experimental.pallas import tpu as pltpu
from jax.experimental.pallas import tpu_sc as plsc

CH = 128
XCH = 64

_SC_PARAMS = pltpu.CompilerParams(needs_layout_passes=False,
                                  use_tc_tiling_on_sc=False)


def _sc_mesh():
    return plsc.VectorSubcoreMesh(core_axis_name="c", subcore_axis_name="s")


def _emb_deg_kernel(V, D, N_pad, K0, K1, NC, NS):
    NW = NC * NS
    rows_w = N_pad // NW
    nx = rows_w // XCH
    rows_s = N_pad // NS

    @functools.partial(
        pl.kernel,
        out_type=[
            jax.ShapeDtypeStruct((N_pad, D), jnp.float32),
            jax.ShapeDtypeStruct((NC, N_pad, 16), jnp.float32),
        ],
        mesh=_sc_mesh(),
        compiler_params=_SC_PARAMS,
        scratch_types=[
            pltpu.VMEM((nx, XCH), jnp.int32),
            pltpu.VMEM((4, XCH, D), jnp.float32),
            pltpu.VMEM((2, 6, CH), jnp.int32),
            pltpu.VMEM((CH, 16), jnp.float32),
            pltpu.VMEM((CH, 16), jnp.float32),
            pltpu.VMEM_SHARED((N_pad, 16), jnp.float32),
            pltpu.SemaphoreType.DMA((4,)),
            pltpu.SemaphoreType.DMA((4,)),
            pltpu.SemaphoreType.DMA((2,)),
            pltpu.SemaphoreType.DMA,
            pltpu.SemaphoreType.DMA,
        ],
    )
    def k(emb_h, xi_h, pk_h, z16_h, h0_h, deg_h,
          xi_v, ebuf, pk_v, w0, w1, deg_sh, egs, ews, pks, ds0, ds1):
        c = lax.axis_index("c")
        s = lax.axis_index("s")
        wid = s * NC + c
        start = jnp.where(c == 0, s * K0, NS * K0 + s * K1)
        cnt = jnp.where(c == 0, K0, K1)
        sl = pl.ds(s * rows_s, rows_s)
        pltpu.sync_copy(z16_h.at[sl], deg_sh.at[sl])
        pltpu.sync_copy(xi_h.at[wid], xi_v)

        def eg(j):
            return pltpu.make_async_copy(emb_h.at[xi_v.at[j]],
                                         ebuf.at[j % 4], egs.at[j % 4])

        def ew(j):
            return pltpu.make_async_copy(
                ebuf.at[j % 4], h0_h.at[pl.ds(wid * rows_w + j * XCH, XCH)],
                ews.at[j % 4])

        def pkd(p, b):
            return pltpu.make_async_copy(pk_h.at[start + p], pk_v.at[b],
                                         pks.at[b])

        def dsc(b, par, wbuf, sem):
            return pltpu.make_async_copy(wbuf, deg_sh.at[pk_v.at[b, 2 + par]],
                                         sem)

        def build(b, par, wbuf):
            bs = lax.broadcast_in_dim(b, (16,), ())
            ws = lax.broadcast_in_dim(4 + par, (16,), ())

            def body(i, carry):
                wi = plsc.load_gather(
                    pk_v, [bs, ws, lax.broadcast_in_dim(i, (16,), ())])
                wbuf[i, :] = plsc.bitcast(wi, jnp.float32)
                return carry

            lax.fori_loop(0, CH, body, 0, unroll=4)

        pkd(0, 0).start()
        for j in range(min(nx, 4)):
            eg(j).start()
        for j in range(nx):
            eg(j).wait()
            ew(j).start()
            if j + 4 < nx:
                ew(j).wait()
                eg(j + 4).start()
        for j in range(max(nx - 4, 0), nx):
            ew(j).wait()

        pkd(0, 0).wait()
        plsc.subcore_barrier()

        def pair(p, carry):
            b = p & 1
            nb = 1 - b

            @pl.when(p + 1 < cnt)
            def _():
                pkd(p + 1, nb).start()

            build(b, 0, w0)
            dsc(b, 0, w0, ds0).start(add=True)
            build(b, 1, w1)
            dsc(b, 1, w1, ds1).start(add=True)

            @pl.when(p + 1 < cnt)
            def _():
                pkd(p + 1, nb).wait()

            dsc(b, 0, w0, ds0).wait()
            dsc(b, 1, w1, ds1).wait()
            return carry

        lax.fori_loop(0, cnt, pair, 0)
        plsc.subcore_barrier()
        pltpu.sync_copy(deg_sh.at[sl], deg_h.at[c].at[sl])

    return k


def _agg_kernel(Dc, N_pad, K0, K1, NC, NS):
    rows_s = N_pad // NS

    @functools.partial(
        pl.kernel,
        out_type=jax.ShapeDtypeStruct((NC, N_pad, Dc), jnp.float32),
        mesh=_sc_mesh(),
        compiler_params=_SC_PARAMS,
        scratch_types=[
            pltpu.VMEM((2, 6, CH), jnp.int32),
            pltpu.VMEM((CH, Dc), jnp.float32),
            pltpu.VMEM((CH, Dc), jnp.float32),
            pltpu.VMEM_SHARED((N_pad, Dc), jnp.float32),
            pltpu.SemaphoreType.DMA((2,)),
            pltpu.SemaphoreType.DMA,
            pltpu.SemaphoreType.DMA,
            pltpu.SemaphoreType.DMA,
            pltpu.SemaphoreType.DMA,
        ],
    )
    def k(g_h, pk_h, acc_h, pk_v, buf0, buf1, acc_sh, pks, gs0, gs1, ss0, ss1):
        c = lax.axis_index("c")
        s = lax.axis_index("s")
        start = jnp.where(c == 0, s * K0, NS * K0 + s * K1)
        cnt = jnp.where(c == 0, K0, K1)
        sl = pl.ds(s * rows_s, rows_s)
        pltpu.sync_copy(g_h.at[sl], acc_sh.at[sl])
        plsc.subcore_barrier()

        def pkd(p, b):
            return pltpu.make_async_copy(pk_h.at[start + p], pk_v.at[b],
                                         pks.at[b])

        def gat(b, par, buf, sem):
            return pltpu.make_async_copy(g_h.at[pk_v.at[b, par]], buf, sem)

        def sca(b, par, buf, sem):
            return pltpu.make_async_copy(buf, acc_sh.at[pk_v.at[b, 2 + par]],
                                         sem)

        def scale(b, par, buf):
            bs = lax.broadcast_in_dim(b, (16,), ())
            ws = lax.broadcast_in_dim(4 + par, (16,), ())

            def body(i, carry):
                wi = plsc.load_gather(
                    pk_v, [bs, ws, lax.broadcast_in_dim(i, (16,), ())])
                wb = plsc.bitcast(wi, jnp.float32)
                for kk in range(Dc // 16):
                    csl = pl.ds(kk * 16, 16)
                    buf[i, csl] = buf[i, csl] * wb
                return carry

            lax.fori_loop(0, CH, body, 0, unroll=2)

        pkd(0, 0).start()
        pkd(0, 0).wait()
        gat(0, 0, buf0, gs0).start()
        gat(0, 1, buf1, gs1).start()

        def pair(p, carry):
            b = p & 1
            nb = 1 - b

            @pl.when(p + 1 < cnt)
            def _():
                pkd(p + 1, nb).start()

            gat(b, 0, buf0, gs0).wait()
            scale(b, 0, buf0)
            sca(b, 0, buf0, ss0).start(add=True)
            gat(b, 1, buf1, gs1).wait()
            scale(b, 1, buf1)
            sca(b, 1, buf1, ss1).start(add=True)

            @pl.when(p + 1 < cnt)
            def _():
                pkd(p + 1, nb).wait()
                sca(b, 0, buf0, ss0).wait()
                gat(nb, 0, buf0, gs0).start()
                sca(b, 1, buf1, ss1).wait()
                gat(nb, 1, buf1, gs1).start()

            @pl.when(p + 1 >= cnt)
            def _():
                sca(b, 0, buf0, ss0).wait()
                sca(b, 1, buf1, ss1).wait()

            return carry

        lax.fori_loop(0, cnt, pair, 0)
        plsc.subcore_barrier()
        pltpu.sync_copy(acc_sh.at[sl], acc_h.at[c].at[sl])

    return k


def _dinv(deg_ref):
    deg = 1.0 + deg_ref[0][:, 0:1] + deg_ref[1][:, 0:1]
    return jnp.where(deg > 0, lax.rsqrt(deg), 0.0)


def _lin1_body(deg_ref, h0_ref, w1_ref, g1_ref):
    dinv = _dinv(deg_ref)
    hw = jnp.dot(h0_ref[...], w1_ref[...], preferred_element_type=jnp.float32)
    g1_ref[...] = dinv * hw


def _lin2_body(acc_ref, g1_ref, deg_ref, b1_ref, w2_ref, g2_ref):
    dinv = _dinv(deg_ref)
    a = acc_ref[0] + acc_ref[1] - g1_ref[...]
    pre = dinv * a + b1_ref[...]
    h1 = jnp.where(pre >= 0, pre, 0.01 * pre)
    hw = jnp.dot(h1, w2_ref[...], preferred_element_type=jnp.float32)
    g2_ref[...] = dinv * hw


def _final_body(acc_ref, g2_ref, deg_ref, b2_ref, out_ref):
    dinv = _dinv(deg_ref)
    z = dinv * (acc_ref[0] + acc_ref[1] - g2_ref[...]) + b2_ref[...]
    valid = lax.broadcasted_iota(jnp.int32, z.shape, 1) < 40
    zm = jnp.where(valid, z, -1e30)
    m = jnp.max(zm, axis=1, keepdims=True)
    e = jnp.where(valid, jnp.exp(zm - m), 0.0)
    ssum = jnp.sum(e, axis=1, keepdims=True)
    out_ref[...] = z - m - jnp.log(ssum)


def kernel(x, edge_index, edge_attr, emb, W1, b1, W2, b2):
    N = x.shape[0]
    E = edge_index.shape[1]
    V, D = emb.shape
    C = W2.shape[1]
    Dc2 = 48

    info = plsc.get_sparse_core_info()
    NC, NS = info.num_cores, info.num_subcores
    NW = NC * NS

    N_pad = -(-N // (NW * XCH)) * (NW * XCH)
    PAIR = 2 * CH
    npl_min = -(-E // PAIR)
    KSUM = -(-npl_min // NS)
    NPL = NS * KSUM
    E_pad = NPL * PAIR

    def _split(fr0):
        k0 = min(KSUM - 1, max(1, round(fr0 * KSUM)))
        return k0, KSUM - k0

    K0e, K1e = _split(0.57)
    K0a, K1a = _split(0.75)
    K0b, K1b = _split(0.63)

    xi = jnp.concatenate([x[:, 0].astype(jnp.int32),
                          jnp.zeros((N_pad - N,), jnp.int32)])
    xi = xi.reshape(NW, (N_pad // NW) // XCH, XCH)
    epad = E_pad - E
    rowp = jnp.concatenate([edge_index[0].astype(jnp.int32),
                            jnp.zeros((epad,), jnp.int32)])
    colp = jnp.concatenate([edge_index[1].astype(jnp.int32),
                            jnp.zeros((epad,), jnp.int32)])
    wp = jnp.concatenate([edge_attr, jnp.zeros((epad,), jnp.float32)])
    wbits = lax.bitcast_convert_type(wp, jnp.int32)
    pk = jnp.concatenate([rowp.reshape(NPL, 2, CH),
                          colp.reshape(NPL, 2, CH),
                          wbits.reshape(NPL, 2, CH)], axis=1)
    z16 = jnp.zeros((N_pad, 16), jnp.float32)

    h0, deg2 = _emb_deg_kernel(V, D, N_pad, K0e, K1e, NC, NS)(emb, xi, pk, z16)

    nblk = N_pad // 512
    deg_spec = pl.BlockSpec((NC, 512, 16), lambda i: (0, i, 0))
    g1 = pl.pallas_call(
        _lin1_body,
        grid=(nblk,),
        in_specs=[deg_spec,
                  pl.BlockSpec((512, D), lambda i: (i, 0)),
                  pl.BlockSpec((D, D), lambda i: (0, 0))],
        out_specs=pl.BlockSpec((512, D), lambda i: (i, 0)),
        out_shape=jax.ShapeDtypeStruct((N_pad, D), jnp.float32),
    )(deg2, h0, W1)

    acc1 = _agg_kernel(D, N_pad, K0a, K1a, NC, NS)(g1, pk)

    W2p = jnp.pad(W2, ((0, 0), (0, Dc2 - C)))
    g2 = pl.pallas_call(
        _lin2_body,
        grid=(nblk,),
        in_specs=[pl.BlockSpec((NC, 512, D), lambda i: (0, i, 0)),
                  pl.BlockSpec((512, D), lambda i: (i, 0)),
                  deg_spec,
                  pl.BlockSpec((1, D), lambda i: (0, 0)),
                  pl.BlockSpec((D, Dc2), lambda i: (0, 0))],
        out_specs=pl.BlockSpec((512, Dc2), lambda i: (i, 0)),
        out_shape=jax.ShapeDtypeStruct((N_pad, Dc2), jnp.float32),
    )(acc1, g1, deg2, b1[None, :], W2p)

    acc2 = _agg_kernel(Dc2, N_pad, K0b, K1b, NC, NS)(g2, pk)

    b2p = jnp.pad(b2, (0, Dc2 - C))
    out = pl.pallas_call(
        _final_body,
        grid=(nblk,),
        in_specs=[pl.BlockSpec((NC, 512, Dc2), lambda i: (0, i, 0)),
                  pl.BlockSpec((512, Dc2), lambda i: (i, 0)),
                  deg_spec,
                  pl.BlockSpec((1, Dc2), lambda i: (0, 0))],
        out_specs=pl.BlockSpec((512, Dc2), lambda i: (i, 0)),
        out_shape=jax.ShapeDtypeStruct((N_pad, Dc2), jnp.float32),
    )(acc2, g2, deg2, b2p[None, :])

    return out[:N, :C]


# speedup vs baseline: 1.4392x; 1.0007x over previous
import functools

import jax
import jax.numpy as jnp
from jax import lax
from jax.experimental import pallas as pl
from jax.experimental.pallas import tpu as pltpu
from jax.experimental.pallas import tpu_sc as plsc

CH = 128
XCH = 64

_SC_PARAMS = pltpu.CompilerParams(needs_layout_passes=False,
                                  use_tc_tiling_on_sc=False)


def _sc_mesh():
    return plsc.VectorSubcoreMesh(core_axis_name="c", subcore_axis_name="s")


def _emb_deg_kernel(V, D, N_pad, K0, K1, NC, NS):
    NW = NC * NS
    rows_w = N_pad // NW
    nx = rows_w // XCH
    rows_s = N_pad // NS

    @functools.partial(
        pl.kernel,
        out_type=[
            jax.ShapeDtypeStruct((N_pad, D), jnp.float32),
            jax.ShapeDtypeStruct((NC, N_pad, 16), jnp.float32),
        ],
        mesh=_sc_mesh(),
        compiler_params=_SC_PARAMS,
        scratch_types=[
            pltpu.VMEM((nx, XCH), jnp.int32),
            pltpu.VMEM((4, XCH, D), jnp.float32),
            pltpu.VMEM((2, 6, CH), jnp.int32),
            pltpu.VMEM((CH, 16), jnp.float32),
            pltpu.VMEM((CH, 16), jnp.float32),
            pltpu.VMEM_SHARED((N_pad, 16), jnp.float32),
            pltpu.SemaphoreType.DMA((4,)),
            pltpu.SemaphoreType.DMA((4,)),
            pltpu.SemaphoreType.DMA((2,)),
            pltpu.SemaphoreType.DMA,
            pltpu.SemaphoreType.DMA,
        ],
    )
    def k(emb_h, xi_h, pk_h, z16_h, h0_h, deg_h,
          xi_v, ebuf, pk_v, w0, w1, deg_sh, egs, ews, pks, ds0, ds1):
        c = lax.axis_index("c")
        s = lax.axis_index("s")
        wid = s * NC + c
        start = jnp.where(c == 0, s * K0, NS * K0 + s * K1)
        cnt = jnp.where(c == 0, K0, K1)
        sl = pl.ds(s * rows_s, rows_s)
        pltpu.sync_copy(z16_h.at[sl], deg_sh.at[sl])
        pltpu.sync_copy(xi_h.at[wid], xi_v)

        def eg(j):
            return pltpu.make_async_copy(emb_h.at[xi_v.at[j]],
                                         ebuf.at[j % 4], egs.at[j % 4])

        def ew(j):
            return pltpu.make_async_copy(
                ebuf.at[j % 4], h0_h.at[pl.ds(wid * rows_w + j * XCH, XCH)],
                ews.at[j % 4])

        def pkd(p, b):
            return pltpu.make_async_copy(pk_h.at[start + p], pk_v.at[b],
                                         pks.at[b])

        def dsc(b, par, wbuf, sem):
            return pltpu.make_async_copy(wbuf, deg_sh.at[pk_v.at[b, 2 + par]],
                                         sem)

        def build(b, par, wbuf):
            bs = lax.broadcast_in_dim(b, (16,), ())
            ws = lax.broadcast_in_dim(4 + par, (16,), ())

            def body(i, carry):
                wi = plsc.load_gather(
                    pk_v, [bs, ws, lax.broadcast_in_dim(i, (16,), ())])
                wbuf[i, :] = plsc.bitcast(wi, jnp.float32)
                return carry

            lax.fori_loop(0, CH, body, 0, unroll=4)

        pkd(0, 0).start()
        for j in range(min(nx, 4)):
            eg(j).start()
        for j in range(nx):
            eg(j).wait()
            ew(j).start()
            if j + 4 < nx:
                ew(j).wait()
                eg(j + 4).start()
        for j in range(max(nx - 4, 0), nx):
            ew(j).wait()

        pkd(0, 0).wait()
        plsc.subcore_barrier()

        def pair(p, carry):
            b = p & 1
            nb = 1 - b

            @pl.when(p + 1 < cnt)
            def _():
                pkd(p + 1, nb).start()

            build(b, 0, w0)
            dsc(b, 0, w0, ds0).start(add=True)
            build(b, 1, w1)
            dsc(b, 1, w1, ds1).start(add=True)

            @pl.when(p + 1 < cnt)
            def _():
                pkd(p + 1, nb).wait()

            dsc(b, 0, w0, ds0).wait()
            dsc(b, 1, w1, ds1).wait()
            return carry

        lax.fori_loop(0, cnt, pair, 0)
        plsc.subcore_barrier()
        pltpu.sync_copy(deg_sh.at[sl], deg_h.at[c].at[sl])

    return k


def _agg_kernel(Dc, N_pad, K0, K1, NC, NS):
    rows_s = N_pad // NS

    @functools.partial(
        pl.kernel,
        out_type=jax.ShapeDtypeStruct((NC, N_pad, Dc), jnp.float32),
        mesh=_sc_mesh(),
        compiler_params=_SC_PARAMS,
        scratch_types=[
            pltpu.VMEM((2, 6, CH), jnp.int32),
            pltpu.VMEM((CH, Dc), jnp.float32),
            pltpu.VMEM((CH, Dc), jnp.float32),
            pltpu.VMEM_SHARED((N_pad, Dc), jnp.float32),
            pltpu.SemaphoreType.DMA((2,)),
            pltpu.SemaphoreType.DMA,
            pltpu.SemaphoreType.DMA,
            pltpu.SemaphoreType.DMA,
            pltpu.SemaphoreType.DMA,
        ],
    )
    def k(g_h, pk_h, acc_h, pk_v, buf0, buf1, acc_sh, pks, gs0, gs1, ss0, ss1):
        c = lax.axis_index("c")
        s = lax.axis_index("s")
        start = jnp.where(c == 0, s * K0, NS * K0 + s * K1)
        cnt = jnp.where(c == 0, K0, K1)
        sl = pl.ds(s * rows_s, rows_s)
        pltpu.sync_copy(g_h.at[sl], acc_sh.at[sl])
        plsc.subcore_barrier()

        def pkd(p, b):
            return pltpu.make_async_copy(pk_h.at[start + p], pk_v.at[b],
                                         pks.at[b])

        def gat(b, par, buf, sem):
            return pltpu.make_async_copy(g_h.at[pk_v.at[b, par]], buf, sem)

        def sca(b, par, buf, sem):
            return pltpu.make_async_copy(buf, acc_sh.at[pk_v.at[b, 2 + par]],
                                         sem)

        def scale(b, par, buf):
            bs = lax.broadcast_in_dim(b, (16,), ())
            ws = lax.broadcast_in_dim(4 + par, (16,), ())

            def body(i, carry):
                wi = plsc.load_gather(
                    pk_v, [bs, ws, lax.broadcast_in_dim(i, (16,), ())])
                wb = plsc.bitcast(wi, jnp.float32)
                for kk in range(Dc // 16):
                    csl = pl.ds(kk * 16, 16)
                    buf[i, csl] = buf[i, csl] * wb
                return carry

            lax.fori_loop(0, CH, body, 0, unroll=4)

        pkd(0, 0).start()
        pkd(0, 0).wait()
        gat(0, 0, buf0, gs0).start()
        gat(0, 1, buf1, gs1).start()

        def pair(p, carry):
            b = p & 1
            nb = 1 - b

            @pl.when(p + 1 < cnt)
            def _():
                pkd(p + 1, nb).start()

            gat(b, 0, buf0, gs0).wait()
            scale(b, 0, buf0)
            sca(b, 0, buf0, ss0).start(add=True)
            gat(b, 1, buf1, gs1).wait()
            scale(b, 1, buf1)
            sca(b, 1, buf1, ss1).start(add=True)

            @pl.when(p + 1 < cnt)
            def _():
                pkd(p + 1, nb).wait()
                sca(b, 0, buf0, ss0).wait()
                gat(nb, 0, buf0, gs0).start()
                sca(b, 1, buf1, ss1).wait()
                gat(nb, 1, buf1, gs1).start()

            @pl.when(p + 1 >= cnt)
            def _():
                sca(b, 0, buf0, ss0).wait()
                sca(b, 1, buf1, ss1).wait()

            return carry

        lax.fori_loop(0, cnt, pair, 0)
        plsc.subcore_barrier()
        pltpu.sync_copy(acc_sh.at[sl], acc_h.at[c].at[sl])

    return k


def _dinv(deg_ref):
    deg = 1.0 + deg_ref[0][:, 0:1] + deg_ref[1][:, 0:1]
    return jnp.where(deg > 0, lax.rsqrt(deg), 0.0)


def _lin1_body(deg_ref, h0_ref, w1_ref, g1_ref):
    dinv = _dinv(deg_ref)
    hw = jnp.dot(h0_ref[...], w1_ref[...], preferred_element_type=jnp.float32)
    g1_ref[...] = dinv * hw


def _lin2_body(acc_ref, g1_ref, deg_ref, b1_ref, w2_ref, g2_ref):
    dinv = _dinv(deg_ref)
    a = acc_ref[0] + acc_ref[1] - g1_ref[...]
    pre = dinv * a + b1_ref[...]
    h1 = jnp.where(pre >= 0, pre, 0.01 * pre)
    hw = jnp.dot(h1, w2_ref[...], preferred_element_type=jnp.float32)
    g2_ref[...] = dinv * hw


def _final_body(acc_ref, g2_ref, deg_ref, b2_ref, out_ref):
    dinv = _dinv(deg_ref)
    z = dinv * (acc_ref[0] + acc_ref[1] - g2_ref[...]) + b2_ref[...]
    valid = lax.broadcasted_iota(jnp.int32, z.shape, 1) < 40
    zm = jnp.where(valid, z, -1e30)
    m = jnp.max(zm, axis=1, keepdims=True)
    e = jnp.where(valid, jnp.exp(zm - m), 0.0)
    ssum = jnp.sum(e, axis=1, keepdims=True)
    out_ref[...] = z - m - jnp.log(ssum)


def kernel(x, edge_index, edge_attr, emb, W1, b1, W2, b2):
    N = x.shape[0]
    E = edge_index.shape[1]
    V, D = emb.shape
    C = W2.shape[1]
    Dc2 = 48

    info = plsc.get_sparse_core_info()
    NC, NS = info.num_cores, info.num_subcores
    NW = NC * NS

    N_pad = -(-N // (NW * XCH)) * (NW * XCH)
    PAIR = 2 * CH
    npl_min = -(-E // PAIR)
    KSUM = -(-npl_min // NS)
    NPL = NS * KSUM
    E_pad = NPL * PAIR

    def _split(fr0):
        k0 = min(KSUM - 1, max(1, round(fr0 * KSUM)))
        return k0, KSUM - k0

    K0e, K1e = _split(0.57)
    K0a, K1a = _split(0.75)
    K0b, K1b = _split(0.63)

    xi = jnp.concatenate([x[:, 0].astype(jnp.int32),
                          jnp.zeros((N_pad - N,), jnp.int32)])
    xi = xi.reshape(NW, (N_pad // NW) // XCH, XCH)
    epad = E_pad - E
    rowp = jnp.concatenate([edge_index[0].astype(jnp.int32),
                            jnp.zeros((epad,), jnp.int32)])
    colp = jnp.concatenate([edge_index[1].astype(jnp.int32),
                            jnp.zeros((epad,), jnp.int32)])
    wp = jnp.concatenate([edge_attr, jnp.zeros((epad,), jnp.float32)])
    wbits = lax.bitcast_convert_type(wp, jnp.int32)
    pk = jnp.concatenate([rowp.reshape(NPL, 2, CH),
                          colp.reshape(NPL, 2, CH),
                          wbits.reshape(NPL, 2, CH)], axis=1)
    z16 = jnp.zeros((N_pad, 16), jnp.float32)

    h0, deg2 = _emb_deg_kernel(V, D, N_pad, K0e, K1e, NC, NS)(emb, xi, pk, z16)

    nblk = N_pad // 512
    deg_spec = pl.BlockSpec((NC, 512, 16), lambda i: (0, i, 0))
    g1 = pl.pallas_call(
        _lin1_body,
        grid=(nblk,),
        in_specs=[deg_spec,
                  pl.BlockSpec((512, D), lambda i: (i, 0)),
                  pl.BlockSpec((D, D), lambda i: (0, 0))],
        out_specs=pl.BlockSpec((512, D), lambda i: (i, 0)),
        out_shape=jax.ShapeDtypeStruct((N_pad, D), jnp.float32),
    )(deg2, h0, W1)

    acc1 = _agg_kernel(D, N_pad, K0a, K1a, NC, NS)(g1, pk)

    W2p = jnp.pad(W2, ((0, 0), (0, Dc2 - C)))
    g2 = pl.pallas_call(
        _lin2_body,
        grid=(nblk,),
        in_specs=[pl.BlockSpec((NC, 512, D), lambda i: (0, i, 0)),
                  pl.BlockSpec((512, D), lambda i: (i, 0)),
                  deg_spec,
                  pl.BlockSpec((1, D), lambda i: (0, 0)),
                  pl.BlockSpec((D, Dc2), lambda i: (0, 0))],
        out_specs=pl.BlockSpec((512, Dc2), lambda i: (i, 0)),
        out_shape=jax.ShapeDtypeStruct((N_pad, Dc2), jnp.float32),
    )(acc1, g1, deg2, b1[None, :], W2p)

    acc2 = _agg_kernel(Dc2, N_pad, K0b, K1b, NC, NS)(g2, pk)

    b2p = jnp.pad(b2, (0, Dc2 - C))
    out = pl.pallas_call(
        _final_body,
        grid=(nblk,),
        in_specs=[pl.BlockSpec((NC, 512, Dc2), lambda i: (0, i, 0)),
                  pl.BlockSpec((512, Dc2), lambda i: (i, 0)),
                  deg_spec,
                  pl.BlockSpec((1, Dc2), lambda i: (0, 0))],
        out_specs=pl.BlockSpec((512, Dc2), lambda i: (i, 0)),
        out_shape=jax.ShapeDtypeStruct((N_pad, Dc2), jnp.float32),
    )(acc2, g2, deg2, b2p[None, :])

    return out[:N, :C]
